# baseline (speedup 1.0000x reference)
import math

import jax
import jax.numpy as jnp
from jax.experimental import pallas as pl

NUM_TO_SAMPLE = 1024
K = 10
_ROWS = 256

_ACOS_C = (
    1.5707963050,
    -0.2145988016,
    0.0889789874,
    -0.0501743046,
    0.0308918810,
    -0.0170881256,
    0.0066700901,
    -0.0012624911,
)


def _fps_body(px_ref, py_ref, pz_ref, out_ref, fmask_ref):
    B, N = px_ref.shape
    px = px_ref[...]
    py = py_ref[...]
    pz = pz_ref[...]
    iota = jax.lax.broadcasted_iota(jnp.int32, (B, N), 1)

    def step(s, state):
        dists, far, fmask = state
        out_ref[pl.ds(s, 1), :] = far[None, :]
        mask = iota == far[:, None]
        fmask = jnp.where(mask, 1.0, fmask)
        cx = jnp.sum(jnp.where(mask, px, 0.0), axis=1)
        cy = jnp.sum(jnp.where(mask, py, 0.0), axis=1)
        cz = jnp.sum(jnp.where(mask, pz, 0.0), axis=1)
        dx = px - cx[:, None]
        dy = py - cy[:, None]
        dz = pz - cz[:, None]
        d = dx * dx + dy * dy + dz * dz
        dists = jnp.minimum(dists, d)
        m = jnp.max(dists, axis=1)
        far_new = jnp.min(
            jnp.where(dists == m[:, None], iota, N), axis=1
        ).astype(jnp.int32)
        return dists, far_new, fmask

    dists0 = jnp.full((B, N), 1e10, dtype=jnp.float32)
    far0 = jnp.zeros((B,), dtype=jnp.int32)
    fmask0 = jnp.zeros((B, N), dtype=jnp.float32)
    _, _, fmask = jax.lax.fori_loop(
        0, NUM_TO_SAMPLE, step, (dists0, far0, fmask0)
    )
    fmask_ref[...] = fmask


def _fps_pallas(pos):
    B, N, _ = pos.shape
    p_idx_t, fmask = pl.pallas_call(
        _fps_body,
        out_shape=(
            jax.ShapeDtypeStruct((NUM_TO_SAMPLE, B), jnp.int32),
            jax.ShapeDtypeStruct((B, N), jnp.float32),
        ),
    )(pos[:, :, 0], pos[:, :, 1], pos[:, :, 2])
    return p_idx_t.T, fmask


def _feat_body(bx_ref, by_ref, bz_ref, xx_ref,
               bxi_ref, byi_ref, bzi_ref, xxi_ref,
               idx_ref):
    R = _ROWS
    N = bx_ref.shape[2]
    bxj = bx_ref[0, :, :]
    byj = by_ref[0, :, :]
    bzj = bz_ref[0, :, :]
    xxj = xx_ref[0, :, :]
    bxi = bxi_ref[0, :, :]
    byi = byi_ref[0, :, :]
    bzi = bzi_ref[0, :, :]
    xxi = xxi_ref[0, :, :]

    m3 = (bxi * bxj + byi * byj) + bzi * bzj
    inner = -2.0 * m3
    key = ((-xxi) - inner) - xxj

    iota = jax.lax.broadcasted_iota(jnp.int32, (R, N), 1).astype(jnp.float32)
    big = jnp.float32(3.4e38)
    nf = jnp.float32(N)
    keyw = key
    for t in range(K):
        m = jnp.max(keyw, axis=1, keepdims=True)
        amin = jnp.min(
            jnp.where(keyw == m, iota, nf), axis=1, keepdims=True
        )
        idx_ref[0, 0, :, t] = amin[:, 0].astype(jnp.int32)
        keyw = jnp.where(iota == amin, -big, keyw)


def _knn_idx(pos):
    B, N, _ = pos.shape
    nb = N // _ROWS
    grid = (B, nb)
    posb = pos.astype(jnp.bfloat16).astype(jnp.float32)
    xx = jnp.sum(jnp.swapaxes(pos, 1, 2) ** 2, axis=1)
    row_spec = pl.BlockSpec((1, 1, N), lambda b, r: (b, 0, 0))
    col_spec = pl.BlockSpec((1, _ROWS, 1), lambda b, r: (b, r, 0))
    out_spec = pl.BlockSpec((1, 1, _ROWS, K), lambda b, r: (b, r, 0, 0))
    idx = pl.pallas_call(
        _feat_body,
        grid=grid,
        in_specs=[row_spec] * 4 + [col_spec] * 4,
        out_specs=out_spec,
        out_shape=jax.ShapeDtypeStruct((B, nb, _ROWS, K), jnp.int32),
    )(posb[:, None, :, 0], posb[:, None, :, 1], posb[:, None, :, 2],
      xx[:, None, :],
      posb[:, :, 0:1], posb[:, :, 1:2], posb[:, :, 2:3],
      xx[:, :, None])
    return idx.reshape(B, N, K)


def _gather1(arr, idx):
    return jax.vmap(lambda a, i: a[i])(arr, idx)


def _safe_norm(d):
    return jnp.sqrt(jnp.sum(d * d, axis=-1) + 1e-12)


def kernel(x, pos, sample_W, sample_b):
    B, N = pos.shape[0], pos.shape[1]
    k = K
    p_idx, fmask = _fps_pallas(jax.lax.stop_gradient(pos))
    fps_feature = (fmask - fmask.mean()) / fmask.sum()
    idxs = _knn_idx(pos)
    xt = jnp.swapaxes(x, 1, 2)
    xn = _gather1(pos, idxs)
    pc = jnp.concatenate([pos, xt], axis=-1)
    ip = jnp.sum(xn * pc[:, :, None, 3:], axis=-1)
    ip = jnp.clip(ip, -1.0, 1.0)
    angle = jnp.arccos(ip)
    thr = math.pi / 2
    angle = jnp.where(angle > thr, math.pi - angle, angle)
    angle = angle.sum(axis=-1)
    curv = (angle - angle.mean()) / angle.sum()
    dists = _safe_norm(xn[..., :3] - pc[:, :, None, :3])
    dmax = dists.max(axis=-1)
    dense = k / (dmax ** 3)
    inf_mask = jnp.isinf(dense)
    max_val = jnp.max(jnp.where(inf_mask, -jnp.inf, dense))
    dense = jnp.where(inf_mask, max_val, dense)
    dense = (dense - dense.mean()) / dense.sum()
    sampling_feats = jnp.stack([fps_feature, curv, dense], axis=-1)
    opt = (sampling_feats @ sample_W.T + sample_b)[..., 0]
    smax = jax.nn.softmax(opt, axis=1)
    topv, topi = jax.lax.top_k(smax, NUM_TO_SAMPLE)
    dist_loss = dmax + dists.mean(axis=-1)
    shorlisted_loss = _gather1(dist_loss, topi)
    sampling_loss = dist_loss * smax
    total_loss = sampling_loss.mean()
    bdist_loss = _gather1(dist_loss, p_idx)
    losses = jnp.stack(
        [total_loss, sampling_loss.mean(), shorlisted_loss.mean(), bdist_loss.mean()]
    )
    return topi, losses


# speedup vs baseline: 2.1743x; 2.1743x over previous
import math

import jax
import jax.numpy as jnp
from jax.experimental import pallas as pl

NUM_TO_SAMPLE = 1024
K = 10
_ROWS = 256

_ACOS_C = (
    1.5707963050,
    -0.2145988016,
    0.0889789874,
    -0.0501743046,
    0.0308918810,
    -0.0170881256,
    0.0066700901,
    -0.0012624911,
)


def _fps_body(px_ref, py_ref, pz_ref, out_ref, fmask_ref):
    B, N = px_ref.shape
    px = px_ref[...]
    py = py_ref[...]
    pz = pz_ref[...]
    iota = jax.lax.broadcasted_iota(jnp.int32, (B, N), 1)

    def step(s, state):
        dists, far, fmask = state
        out_ref[pl.ds(s, 1), :] = far[None, :]
        mask = iota == far[:, None]
        fmask = jnp.where(mask, 1.0, fmask)
        cx = jnp.sum(jnp.where(mask, px, 0.0), axis=1)
        cy = jnp.sum(jnp.where(mask, py, 0.0), axis=1)
        cz = jnp.sum(jnp.where(mask, pz, 0.0), axis=1)
        dx = px - cx[:, None]
        dy = py - cy[:, None]
        dz = pz - cz[:, None]
        d = dx * dx + dy * dy + dz * dz
        dists = jnp.minimum(dists, d)
        m = jnp.max(dists, axis=1)
        far_new = jnp.min(
            jnp.where(dists == m[:, None], iota, N), axis=1
        ).astype(jnp.int32)
        return dists, far_new, fmask

    dists0 = jnp.full((B, N), 1e10, dtype=jnp.float32)
    far0 = jnp.zeros((B,), dtype=jnp.int32)
    fmask0 = jnp.zeros((B, N), dtype=jnp.float32)
    _, _, fmask = jax.lax.fori_loop(
        0, NUM_TO_SAMPLE, step, (dists0, far0, fmask0)
    )
    fmask_ref[...] = fmask


def _fps_pallas(pos):
    B, N, _ = pos.shape
    p_idx_t, fmask = pl.pallas_call(
        _fps_body,
        out_shape=(
            jax.ShapeDtypeStruct((NUM_TO_SAMPLE, B), jnp.int32),
            jax.ShapeDtypeStruct((B, N), jnp.float32),
        ),
    )(pos[:, :, 0], pos[:, :, 1], pos[:, :, 2])
    return p_idx_t.T, fmask


def _feat_body(px_ref, py_ref, pz_ref, bx_ref, by_ref, bz_ref, xx_ref,
               pxi_ref, pyi_ref, pzi_ref, bxi_ref, byi_ref, bzi_ref,
               xxi_ref, xf0_ref, xf1_ref, xf2_ref,
               ip_ref, d2_ref):
    R = _ROWS
    N = px_ref.shape[2]
    pxj = px_ref[0, :, :]
    pyj = py_ref[0, :, :]
    pzj = pz_ref[0, :, :]
    bxj = bx_ref[0, :, :]
    byj = by_ref[0, :, :]
    bzj = bz_ref[0, :, :]
    xxj = xx_ref[0, :, :]
    pxi = pxi_ref[0, :, :]
    pyi = pyi_ref[0, :, :]
    pzi = pzi_ref[0, :, :]
    bxi = bxi_ref[0, :, :]
    byi = byi_ref[0, :, :]
    bzi = bzi_ref[0, :, :]
    xxi = xxi_ref[0, :, :]
    xf0 = xf0_ref[0, :, :]
    xf1 = xf1_ref[0, :, :]
    xf2 = xf2_ref[0, :, :]

    dx = pxi - pxj
    dy = pyi - pyj
    dz = pzi - pzj
    d2 = dx * dx + dy * dy + dz * dz
    g = xf0 * pxj + xf1 * pyj + xf2 * pzj
    m3 = (bxi * bxj + byi * byj) + bzi * bzj
    inner = -2.0 * m3
    key = ((-xxi) - inner) - xxj

    iota = jax.lax.broadcasted_iota(jnp.int32, (R, N), 1).astype(jnp.float32)
    big = jnp.float32(3.4e38)
    nf = jnp.float32(N)
    keyw = key
    for t in range(K):
        m = jnp.max(keyw, axis=1, keepdims=True)
        amin = jnp.min(
            jnp.where(keyw == m, iota, nf), axis=1, keepdims=True
        )
        sel = iota == amin
        ip_ref[0, 0, :, t] = jnp.sum(jnp.where(sel, g, 0.0), axis=1)
        d2_ref[0, 0, :, t] = jnp.sum(jnp.where(sel, d2, 0.0), axis=1)
        keyw = jnp.where(sel, -big, keyw)


def _knn_feats(pos, x):
    B, N, _ = pos.shape
    nb = N // _ROWS
    grid = (B, nb)
    posb = pos.astype(jnp.bfloat16).astype(jnp.float32)
    xx = jnp.sum(jnp.swapaxes(pos, 1, 2) ** 2, axis=1)
    row_spec = pl.BlockSpec((1, 1, N), lambda b, r: (b, 0, 0))
    col_spec = pl.BlockSpec((1, _ROWS, 1), lambda b, r: (b, r, 0))
    out_spec = pl.BlockSpec((1, 1, _ROWS, K), lambda b, r: (b, r, 0, 0))
    oshape = jax.ShapeDtypeStruct((B, nb, _ROWS, K), jnp.float32)
    ip, d2 = pl.pallas_call(
        _feat_body,
        grid=grid,
        in_specs=[row_spec] * 7 + [col_spec] * 10,
        out_specs=(out_spec, out_spec),
        out_shape=(oshape, oshape),
    )(pos[:, None, :, 0], pos[:, None, :, 1], pos[:, None, :, 2],
      posb[:, None, :, 0], posb[:, None, :, 1], posb[:, None, :, 2],
      xx[:, None, :],
      pos[:, :, 0:1], pos[:, :, 1:2], pos[:, :, 2:3],
      posb[:, :, 0:1], posb[:, :, 1:2], posb[:, :, 2:3],
      xx[:, :, None],
      x[:, 0, :, None], x[:, 1, :, None], x[:, 2, :, None])
    return ip.reshape(B, N, K), d2.reshape(B, N, K)


def _gather1(arr, idx):
    return jax.vmap(lambda a, i: a[i])(arr, idx)


def kernel(x, pos, sample_W, sample_b):
    B, N = pos.shape[0], pos.shape[1]
    k = K
    p_idx, fmask = _fps_pallas(jax.lax.stop_gradient(pos))
    fps_feature = (fmask - fmask.mean()) / fmask.sum()
    ip, d2 = _knn_feats(pos, x)
    ip = jnp.clip(ip, -1.0, 1.0)
    angle = jnp.arccos(ip)
    thr = math.pi / 2
    angle = jnp.where(angle > thr, math.pi - angle, angle)
    angle = angle.sum(axis=-1)
    curv = (angle - angle.mean()) / angle.sum()
    dists = jnp.sqrt(d2 + 1e-12)
    dmax = dists.max(axis=-1)
    dense = k / (dmax ** 3)
    inf_mask = jnp.isinf(dense)
    max_val = jnp.max(jnp.where(inf_mask, -jnp.inf, dense))
    dense = jnp.where(inf_mask, max_val, dense)
    dense = (dense - dense.mean()) / dense.sum()
    sampling_feats = jnp.stack([fps_feature, curv, dense], axis=-1)
    opt = (sampling_feats @ sample_W.T + sample_b)[..., 0]
    smax = jax.nn.softmax(opt, axis=1)
    topv, topi = jax.lax.top_k(smax, NUM_TO_SAMPLE)
    dist_loss = dmax + dists.mean(axis=-1)
    shorlisted_loss = _gather1(dist_loss, topi)
    sampling_loss = dist_loss * smax
    total_loss = sampling_loss.mean()
    bdist_loss = _gather1(dist_loss, p_idx)
    losses = jnp.stack(
        [total_loss, sampling_loss.mean(), shorlisted_loss.mean(), bdist_loss.mean()]
    )
    return topi, losses
